# baseline (speedup 1.0000x reference)
import jax
import jax.numpy as jnp
from jax.experimental import pallas as pl
from jax.experimental.pallas import tpu as pltpu

_B = 16384
_A = 26
_V = 128
_ROWS = 256

_C0 = 9.975032552085419e-06
_C1 = 0.9992354838332742
_C2 = -0.4902307234234026
_C3 = 0.28527268109056003
_C4 = -0.1315818250887567
_C5 = 0.03044900453867246


def _loss_kernel(si_ref, ro_ref, loss_ref, acc_ref, accor_ref):
    x = ro_ref[...]
    si = si_ref[...]
    x3 = x.reshape(_ROWS, _A, _V)

    u = jnp.exp(-jnp.abs(x))
    p = _C5
    for c in (_C4, _C3, _C2, _C1, _C0):
        p = p * u + c
    sp = jnp.maximum(x, 0.0) + p

    iota_v = jax.lax.broadcasted_iota(jnp.int32, (_ROWS, _A, _V), 2)
    onehot = iota_v == si[:, :, None]
    contrib = jnp.where(onehot, sp.reshape(_ROWS, _A, _V) - x3, sp.reshape(_ROWS, _A, _V))
    s_loss = jnp.sum(contrib)

    m = jnp.max(x3, axis=2, keepdims=True)
    idx = jnp.min(jnp.where(x3 == m, iota_v, _V), axis=2)
    correct = idx == si
    s_accor = jnp.sum(correct.astype(jnp.float32))
    s_acc = jnp.sum((jnp.sum(correct.astype(jnp.int32), axis=1) == _A)
                    .astype(jnp.float32))

    loss_ref[...] = s_loss.reshape(1, 1, 1)
    acc_ref[...] = s_acc.reshape(1, 1, 1)
    accor_ref[...] = s_accor.reshape(1, 1, 1)


def kernel(sender_input, _message, _receiver_input, receiver_output, _labels):
    n_blocks = _B // _ROWS
    out_shape = [jax.ShapeDtypeStruct((n_blocks, 1, 1), jnp.float32)] * 3
    loss_p, acc_p, accor_p = pl.pallas_call(
        _loss_kernel,
        grid=(n_blocks,),
        in_specs=[
            pl.BlockSpec((_ROWS, _A), lambda i: (i, 0)),
            pl.BlockSpec((_ROWS, _A * _V), lambda i: (i, 0)),
        ],
        out_specs=[pl.BlockSpec((1, 1, 1), lambda i: (i, 0, 0))] * 3,
        out_shape=out_shape,
        compiler_params=pltpu.CompilerParams(
            dimension_semantics=("parallel",)),
    )(sender_input, receiver_output)
    denom = jnp.float32(_B * _A * _V)
    loss = jnp.sum(loss_p) / denom
    acc = jnp.sum(acc_p) / jnp.float32(_B)
    acc_or = jnp.sum(accor_p) / jnp.float32(_B * _A)
    return (loss, acc, acc_or)
